# baseline (speedup 1.0000x reference)
import functools

import jax
import jax.numpy as jnp
from jax import lax
from jax.experimental import pallas as pl
from jax.experimental.pallas import tpu as pltpu
from jax.experimental.pallas import tpu_sc as plsc

N = 10000
F = 16
FE = 4
E = 320000
H = 64
LAT = 128
FCH = 256
L = 2

NC = 2
NS = 16
NW = NC * NS


_G_ROWS = 2 * E
_G_PER_W = _G_ROWS // NW
_G_CH = 128
_G_K = 2
_G_GROUP = _G_CH * _G_K
_G_NGRP = _G_PER_W // _G_GROUP
_G_TAIL = _G_PER_W - _G_NGRP * _G_GROUP


_G_STRIPE = N // NS


def _gather_body(x_hbm, idx_hbm, out_hbm, idx_v, rows_v, tail_idx_v, tail_rows_v,
                 x_shared, gsem, osem):
    sid = lax.axis_index("s")
    wid = sid * NC + lax.axis_index("c")
    base = wid * _G_PER_W
    pltpu.sync_copy(x_hbm.at[pl.ds(sid * _G_STRIPE, _G_STRIPE)],
                    x_shared.at[pl.ds(sid * _G_STRIPE, _G_STRIPE)])
    pltpu.sync_copy(idx_hbm.at[pl.ds(base, _G_PER_W)], idx_v)
    plsc.subcore_barrier()

    @pl.loop(0, _G_NGRP, step=2)
    def _group(g0):
        for b in range(2):
            g = g0 + b
            off = g * _G_GROUP
            descs = [
                pltpu.async_copy(
                    x_shared.at[idx_v.at[pl.ds(off + j * _G_CH, _G_CH)]],
                    rows_v.at[b, pl.ds(j * _G_CH, _G_CH)], gsem)
                for j in range(_G_K)
            ]
            for d in descs:
                d.wait()
            @pl.when(g > 0)
            def _():
                pltpu.make_async_copy(
                    rows_v.at[1 - b], out_hbm.at[pl.ds(0, _G_GROUP)],
                    osem).wait()
            pltpu.async_copy(rows_v.at[b],
                             out_hbm.at[pl.ds(base + off, _G_GROUP)], osem)

    toff = base + _G_NGRP * _G_GROUP
    pltpu.sync_copy(idx_hbm.at[pl.ds(toff, _G_TAIL)], tail_idx_v)
    pltpu.async_copy(x_shared.at[tail_idx_v], tail_rows_v, gsem).wait()
    pltpu.sync_copy(tail_rows_v, out_hbm.at[pl.ds(toff, _G_TAIL)])
    pltpu.make_async_copy(rows_v.at[0], out_hbm.at[pl.ds(0, _G_GROUP)],
                          osem).wait()


@functools.cache
def _sc_mesh():
    return plsc.VectorSubcoreMesh(
        core_axis_name="c", subcore_axis_name="s",
        num_cores=NC, num_subcores=NS)


_SC_PARAMS = pltpu.CompilerParams(use_tc_tiling_on_sc=False)


@functools.cache
def _gather_kernel():
    return pl.kernel(
        _gather_body,
        out_type=jax.ShapeDtypeStruct((_G_ROWS, F), jnp.float32),
        mesh=_sc_mesh(),
        compiler_params=_SC_PARAMS,
        scratch_types=[
            pltpu.VMEM((_G_PER_W,), jnp.int32),
            pltpu.VMEM((2, _G_GROUP, F), jnp.float32),
            pltpu.VMEM((_G_TAIL,), jnp.int32),
            pltpu.VMEM((_G_TAIL, F), jnp.float32),
            pltpu.VMEM_SHARED((N, F), jnp.float32),
            pltpu.SemaphoreType.DMA,
            pltpu.SemaphoreType.DMA,
        ],
    )


def _gather(x, idx):
    return _gather_kernel()(x, idx)


_S_CH = 128
_S_ROWS_PW = 78
_S_MAIN_PW = _S_ROWS_PW * _S_CH
_S_HK = _S_ROWS_PW // 2
_S_HALF = _S_HK * _S_CH
_S_TAIL = 16
_S_TAIL_BASE = NW * _S_MAIN_PW
_S_ZROWS = N // NS


def _scatter_body(msg_hbm, dst2d_hbm, out_hbm, idx2_v, msg_v,
                  tidx_v, tmsg_v, stripe_v, acc_shared, sem):
    cid = lax.axis_index("c")
    sid = lax.axis_index("s")
    wid = cid * NS + sid
    def zrow(j, _):
        stripe_v[j, :] = jnp.zeros((F,), jnp.float32)
        return _
    lax.fori_loop(0, _S_ZROWS, zrow, 0, unroll=8)
    pltpu.sync_copy(stripe_v, acc_shared.at[pl.ds(sid * _S_ZROWS, _S_ZROWS)])
    plsc.subcore_barrier()

    pltpu.sync_copy(dst2d_hbm.at[pl.ds(wid * _S_ROWS_PW, _S_ROWS_PW)], idx2_v)
    for half in range(2):
        eoff = wid * _S_MAIN_PW + half * _S_HALF
        pltpu.sync_copy(msg_hbm.at[pl.ds(eoff, _S_HALF)], msg_v)
        descs = [
            pltpu.async_copy(msg_v.at[pl.ds(k * _S_CH, _S_CH)],
                             acc_shared.at[idx2_v.at[half * _S_HK + k]],
                             sem, add=True)
            for k in range(_S_HK)
        ]
        for d in descs:
            d.wait()
    toff = _S_TAIL_BASE + wid * _S_TAIL
    pltpu.sync_copy(
        dst2d_hbm.at[_S_TAIL_BASE // _S_CH + wid // 8,
                     pl.ds((wid % 8) * _S_TAIL, _S_TAIL)], tidx_v)
    pltpu.sync_copy(msg_hbm.at[pl.ds(toff, _S_TAIL)], tmsg_v)
    pltpu.sync_copy(tmsg_v, acc_shared.at[tidx_v], add=True)
    plsc.subcore_barrier()
    pltpu.sync_copy(acc_shared.at[pl.ds(sid * _S_ZROWS, _S_ZROWS)], stripe_v)
    pltpu.sync_copy(stripe_v,
                    out_hbm.at[pl.ds(cid * N + sid * _S_ZROWS, _S_ZROWS)])


@functools.cache
def _scatter_kernel():
    return pl.kernel(
        _scatter_body,
        out_type=jax.ShapeDtypeStruct((2 * N, F), jnp.float32),
        mesh=_sc_mesh(),
        compiler_params=_SC_PARAMS,
        scratch_types=[
            pltpu.VMEM((_S_ROWS_PW, _S_CH), jnp.int32),
            pltpu.VMEM((_S_HALF, F), jnp.float32),
            pltpu.VMEM((_S_TAIL,), jnp.int32),
            pltpu.VMEM((_S_TAIL, F), jnp.float32),
            pltpu.VMEM((_S_ZROWS, F), jnp.float32),
            pltpu.VMEM_SHARED((N, F), jnp.float32),
            pltpu.SemaphoreType.DMA,
        ],
    )


def _scatter(msg, dst2d):
    return _scatter_kernel()(msg, dst2d)


_PK = 8
_BR = 800
_EROWS = E // _PK
_GROWS = _G_ROWS // _PK


def _silu(v):
    return v * jax.nn.sigmoid(v)


def _bf(v):
    return v.astype(jnp.bfloat16)


def _edge_mlp_body(xs_ref, xd_ref, ea_ref, w1s, w1d, w1e, b1, w2, b2, out_ref):
    ea32 = ea_ref[...]
    ea8 = jnp.concatenate(
        [ea32[:, 32 * t:32 * (t + 1)] for t in range(4)], axis=0)
    h = (jnp.dot(_bf(xs_ref[...]), w1s[...], preferred_element_type=jnp.float32)
         + jnp.dot(_bf(xd_ref[...]), w1d[...], preferred_element_type=jnp.float32)
         + jnp.dot(_bf(ea8), w1e[...], preferred_element_type=jnp.float32)
         + b1[...])
    out_ref[...] = jnp.dot(_bf(_silu(h)), w2[...],
                           preferred_element_type=jnp.float32) + b2[...]


def _edge_mlp(g_p, eap_p, w1s, w1d, w1e, b1, w2, b2):
    grid = _EROWS // _BR
    wspec = lambda shape: pl.BlockSpec(shape, lambda i: (0, 0))
    return pl.pallas_call(
        _edge_mlp_body,
        grid=(grid,),
        in_specs=[
            pl.BlockSpec((_BR, 128), lambda i: (i, 0)),
            pl.BlockSpec((_BR, 128), lambda i: (i + grid, 0)),
            pl.BlockSpec((_BR // 4, 128), lambda i: (i, 0)),
            wspec((128, _PK * H)), wspec((128, _PK * H)),
            wspec((_PK * FE, _PK * H)),
            wspec((1, _PK * H)), wspec((_PK * H, 128)), wspec((1, 128)),
        ],
        out_specs=pl.BlockSpec((_BR, 128), lambda i: (i, 0)),
        out_shape=jax.ShapeDtypeStruct((_EROWS, 128), jnp.float32),
    )(g_p, g_p, eap_p, w1s, w1d, w1e, b1, w2, b2)


_NP = N // _PK


def _update_body(x_ref, aggp_ref, w1x, w1a, b1, w2, b2, out_ref):
    agg = aggp_ref[0:_NP, :] + aggp_ref[_NP:2 * _NP, :]
    h = (jnp.dot(_bf(x_ref[...]), w1x[...], preferred_element_type=jnp.float32)
         + jnp.dot(_bf(agg), w1a[...], preferred_element_type=jnp.float32)
         + b1[...])
    out_ref[...] = x_ref[...] + jnp.dot(
        _bf(_silu(h)), w2[...], preferred_element_type=jnp.float32) + b2[...]


def _update(x_p, aggp_p, w1x, w1a, b1, w2, b2):
    return pl.pallas_call(
        _update_body,
        out_shape=jax.ShapeDtypeStruct((_NP, 128), jnp.float32),
    )(x_p, aggp_p, w1x, w1a, b1, w2, b2)


_FC_BK = 6400


def _fc1_body(flat_ref, w1_ref, b1_ref, out_ref):
    k = pl.program_id(0)
    part = jnp.dot(flat_ref[...], w1_ref[...], preferred_element_type=jnp.float32)

    @pl.when(k == 0)
    def _():
        out_ref[...] = part + b1_ref[...]

    @pl.when(k > 0)
    def _():
        out_ref[...] += part


def _fc1(flat, w1, b1):
    grid = (N * F) // _FC_BK
    return pl.pallas_call(
        _fc1_body,
        grid=(grid,),
        in_specs=[
            pl.BlockSpec((1, _FC_BK), lambda k: (0, k)),
            pl.BlockSpec((_FC_BK, FCH), lambda k: (k, 0)),
            pl.BlockSpec((1, FCH), lambda k: (0, 0)),
        ],
        out_specs=pl.BlockSpec((1, FCH), lambda k: (0, 0)),
        out_shape=jax.ShapeDtypeStruct((1, FCH), jnp.float32),
    )(flat, w1, b1)


def _fc_mid_body(s1_ref, ew2, eb2, dw1, db1, out_ref):
    z = jnp.dot(_silu(s1_ref[...]), ew2[...],
                preferred_element_type=jnp.float32) + eb2[...]
    out_ref[...] = _silu(
        jnp.dot(z, dw1[...], preferred_element_type=jnp.float32) + db1[...])


def _fc_mid(s1, ew2, eb2, dw1, db1):
    return pl.pallas_call(
        _fc_mid_body,
        out_shape=jax.ShapeDtypeStruct((1, FCH), jnp.float32),
    )(s1, ew2, eb2, dw1, db1)


_FC_BN = 6400


def _fc2_body(t_ref, w2_ref, b2_ref, out_ref):
    out_ref[...] = jnp.dot(t_ref[...], w2_ref[...],
                           preferred_element_type=jnp.float32) + b2_ref[...]


def _fc2(t, w2, b2):
    grid = (N * F) // _FC_BN
    return pl.pallas_call(
        _fc2_body,
        grid=(grid,),
        in_specs=[
            pl.BlockSpec((1, FCH), lambda j: (0, 0)),
            pl.BlockSpec((FCH, _FC_BN), lambda j: (0, j)),
            pl.BlockSpec((1, _FC_BN), lambda j: (0, j)),
        ],
        out_specs=pl.BlockSpec((1, _FC_BN), lambda j: (0, j)),
        out_shape=jax.ShapeDtypeStruct((1, N * F), jnp.float32),
    )(t, w2, b2)


def _kron8(w):
    return _bf(jnp.kron(jnp.eye(_PK, dtype=jnp.float32), w))


def _tile8(b):
    return jnp.tile(b, _PK).reshape(1, _PK * b.shape[0])


def _block(h_p, idx2e, dst2d, ea8, mW1, mb1, mW2, mb2, uW1, ub1, uW2, ub2):
    g = _gather(h_p.reshape(N, F), idx2e)
    msg_p = _edge_mlp(g.reshape(_GROWS, 128), ea8,
                      _kron8(mW1[:F]), _kron8(mW1[F:2 * F]),
                      _kron8(mW1[2 * F:]),
                      _tile8(mb1), _kron8(mW2), _tile8(mb2))
    aggp = _scatter(msg_p.reshape(E, F), dst2d)
    return _update(h_p, aggp.reshape(2 * _NP, 128),
                   _kron8(uW1[:F]), _kron8(uW1[F:2 * F]), _tile8(ub1),
                   _kron8(uW2), _tile8(ub2))


def kernel(x, edge_index, edge_attr, enc_mW1, enc_mb1, enc_mW2, enc_mb2,
           enc_uW1, enc_ub1, enc_uW2, enc_ub2, dec_mW1, dec_mb1, dec_mW2,
           dec_mb2, dec_uW1, dec_ub1, dec_uW2, dec_ub2, fcE_W1, fcE_b1,
           fcE_W2, fcE_b2, fcD_W1, fcD_b1, fcD_W2, fcD_b2):
    def _permq(a):
        return (a.reshape(E // (_PK * _BR), _BR // 4, 4, _PK)
                .transpose(0, 2, 1, 3).reshape(E))

    src_p = _permq(edge_index[0])
    dst_p = _permq(edge_index[1])
    idx2e = jnp.concatenate([src_p, dst_p])
    dst2d = dst_p.reshape(E // _S_CH, _S_CH)
    ea8 = edge_attr.reshape(E // 32, 128)

    h_p = x.reshape(_NP, 128)
    for l in range(L):
        h_p = _block(h_p, idx2e, dst2d, ea8, enc_mW1[l], enc_mb1[l],
                     enc_mW2[l], enc_mb2[l], enc_uW1[l], enc_ub1[l],
                     enc_uW2[l], enc_ub2[l])
    s1 = _fc1(h_p.reshape(1, N * F), fcE_W1, fcE_b1.reshape(1, FCH))
    t = _fc_mid(s1, fcE_W2, fcE_b2.reshape(1, LAT), fcD_W1,
                fcD_b1.reshape(1, FCH))
    d = _fc2(t, fcD_W2, fcD_b2.reshape(1, N * F))
    h_p = d.reshape(_NP, 128)
    for l in range(L):
        h_p = _block(h_p, idx2e, dst2d, ea8, dec_mW1[l], dec_mb1[l],
                     dec_mW2[l], dec_mb2[l], dec_uW1[l], dec_ub1[l],
                     dec_uW2[l], dec_ub2[l])
    return h_p.reshape(N, F)


# speedup vs baseline: 1.1128x; 1.1128x over previous
import functools

import jax
import jax.numpy as jnp
from jax import lax
from jax.experimental import pallas as pl
from jax.experimental.pallas import tpu as pltpu
from jax.experimental.pallas import tpu_sc as plsc

N = 10000
F = 16
FE = 4
E = 320000
H = 64
LAT = 128
FCH = 256
L = 2

NC = 2
NS = 16
NW = NC * NS


_G_ROWS = 2 * E
_G_PER_W = _G_ROWS // NW
_G_CH = 128
_G_K = 2
_G_GROUP = _G_CH * _G_K
_G_NGRP = _G_PER_W // _G_GROUP
_G_TAIL = _G_PER_W - _G_NGRP * _G_GROUP


_G_STRIPE = N // NS


def _gather_body(x_hbm, idx_hbm, out_hbm, idx_v, rows_v, tail_idx_v, tail_rows_v,
                 x_shared, gsem, osem):
    sid = lax.axis_index("s")
    wid = sid * NC + lax.axis_index("c")
    base = wid * _G_PER_W
    pltpu.sync_copy(x_hbm.at[pl.ds(sid * _G_STRIPE, _G_STRIPE)],
                    x_shared.at[pl.ds(sid * _G_STRIPE, _G_STRIPE)])
    pltpu.sync_copy(idx_hbm.at[pl.ds(base, _G_PER_W)], idx_v)
    plsc.subcore_barrier()

    @pl.loop(0, _G_NGRP, step=2)
    def _group(g0):
        for b in range(2):
            g = g0 + b
            off = g * _G_GROUP
            descs = [
                pltpu.async_copy(
                    x_shared.at[idx_v.at[pl.ds(off + j * _G_CH, _G_CH)]],
                    rows_v.at[b, pl.ds(j * _G_CH, _G_CH)], gsem)
                for j in range(_G_K)
            ]
            for d in descs:
                d.wait()
            @pl.when(g > 0)
            def _():
                pltpu.make_async_copy(
                    rows_v.at[1 - b], out_hbm.at[pl.ds(0, _G_GROUP)],
                    osem).wait()
            pltpu.async_copy(rows_v.at[b],
                             out_hbm.at[pl.ds(base + off, _G_GROUP)], osem)

    toff = base + _G_NGRP * _G_GROUP
    pltpu.sync_copy(idx_hbm.at[pl.ds(toff, _G_TAIL)], tail_idx_v)
    pltpu.async_copy(x_shared.at[tail_idx_v], tail_rows_v, gsem).wait()
    pltpu.sync_copy(tail_rows_v, out_hbm.at[pl.ds(toff, _G_TAIL)])
    pltpu.make_async_copy(rows_v.at[0], out_hbm.at[pl.ds(0, _G_GROUP)],
                          osem).wait()


@functools.cache
def _sc_mesh():
    return plsc.VectorSubcoreMesh(
        core_axis_name="c", subcore_axis_name="s",
        num_cores=NC, num_subcores=NS)


_SC_PARAMS = pltpu.CompilerParams(use_tc_tiling_on_sc=False)


@functools.cache
def _gather_kernel():
    return pl.kernel(
        _gather_body,
        out_type=jax.ShapeDtypeStruct((_G_ROWS, F), jnp.float32),
        mesh=_sc_mesh(),
        compiler_params=_SC_PARAMS,
        scratch_types=[
            pltpu.VMEM((_G_PER_W,), jnp.int32),
            pltpu.VMEM((2, _G_GROUP, F), jnp.float32),
            pltpu.VMEM((_G_TAIL,), jnp.int32),
            pltpu.VMEM((_G_TAIL, F), jnp.float32),
            pltpu.VMEM_SHARED((N, F), jnp.float32),
            pltpu.SemaphoreType.DMA,
            pltpu.SemaphoreType.DMA,
        ],
    )


def _gather(x, idx):
    return _gather_kernel()(x, idx)


_S_CH = 128
_S_ROWS_PW = 78
_S_MAIN_PW = _S_ROWS_PW * _S_CH
_S_HK = _S_ROWS_PW // 2
_S_HALF = _S_HK * _S_CH
_S_TAIL = 16
_S_TAIL_BASE = NW * _S_MAIN_PW
_S_ZROWS = N // NS


def _scatter_body(msg_hbm, dst2d_hbm, out_hbm, idx2_v, msg_v,
                  tidx_v, tmsg_v, stripe_v, acc_shared, sem):
    cid = lax.axis_index("c")
    sid = lax.axis_index("s")
    wid = cid * NS + sid
    def zrow(j, _):
        stripe_v[j, :] = jnp.zeros((F,), jnp.float32)
        return _
    lax.fori_loop(0, _S_ZROWS, zrow, 0, unroll=8)
    pltpu.sync_copy(stripe_v, acc_shared.at[pl.ds(sid * _S_ZROWS, _S_ZROWS)])
    plsc.subcore_barrier()

    pltpu.sync_copy(dst2d_hbm.at[pl.ds(wid * _S_ROWS_PW, _S_ROWS_PW)], idx2_v)
    for half in range(2):
        eoff = wid * _S_MAIN_PW + half * _S_HALF
        pltpu.sync_copy(msg_hbm.at[pl.ds(eoff, _S_HALF)], msg_v)
        descs = [
            pltpu.async_copy(msg_v.at[pl.ds(k * _S_CH, _S_CH)],
                             acc_shared.at[idx2_v.at[half * _S_HK + k]],
                             sem, add=True)
            for k in range(_S_HK)
        ]
        for d in descs:
            d.wait()
    toff = _S_TAIL_BASE + wid * _S_TAIL
    pltpu.sync_copy(
        dst2d_hbm.at[_S_TAIL_BASE // _S_CH + wid // 8,
                     pl.ds((wid % 8) * _S_TAIL, _S_TAIL)], tidx_v)
    pltpu.sync_copy(msg_hbm.at[pl.ds(toff, _S_TAIL)], tmsg_v)
    pltpu.sync_copy(tmsg_v, acc_shared.at[tidx_v], add=True)
    plsc.subcore_barrier()
    pltpu.sync_copy(acc_shared.at[pl.ds(sid * _S_ZROWS, _S_ZROWS)], stripe_v)
    pltpu.sync_copy(stripe_v,
                    out_hbm.at[pl.ds(cid * N + sid * _S_ZROWS, _S_ZROWS)])


@functools.cache
def _scatter_kernel():
    return pl.kernel(
        _scatter_body,
        out_type=jax.ShapeDtypeStruct((2 * N, F), jnp.float32),
        mesh=_sc_mesh(),
        compiler_params=_SC_PARAMS,
        scratch_types=[
            pltpu.VMEM((_S_ROWS_PW, _S_CH), jnp.int32),
            pltpu.VMEM((_S_HALF, F), jnp.float32),
            pltpu.VMEM((_S_TAIL,), jnp.int32),
            pltpu.VMEM((_S_TAIL, F), jnp.float32),
            pltpu.VMEM((_S_ZROWS, F), jnp.float32),
            pltpu.VMEM_SHARED((N, F), jnp.float32),
            pltpu.SemaphoreType.DMA,
        ],
    )


def _scatter(msg, dst2d):
    return _scatter_kernel()(msg, dst2d)


_PK = 8
_BR = 800
_EROWS = E // _PK
_GROWS = _G_ROWS // _PK


def _silu(v):
    return v * jax.nn.sigmoid(v)


def _bf(v):
    return v.astype(jnp.bfloat16)


def _edge_mlp_body(xs_ref, xd_ref, ea_ref, w1s, w1d, w1e, b1, w2, b2, out_ref):
    ea32 = ea_ref[...]
    ea8 = jnp.stack(
        [ea32[:, 32 * t:32 * (t + 1)] for t in range(4)],
        axis=1).reshape(_BR, _PK * FE)
    h = (jnp.dot(_bf(xs_ref[...]), w1s[...], preferred_element_type=jnp.float32)
         + jnp.dot(_bf(xd_ref[...]), w1d[...], preferred_element_type=jnp.float32)
         + jnp.dot(_bf(ea8), w1e[...], preferred_element_type=jnp.float32)
         + b1[...])
    out_ref[...] = jnp.dot(_bf(_silu(h)), w2[...],
                           preferred_element_type=jnp.float32) + b2[...]


def _edge_mlp(g_p, eap_p, w1s, w1d, w1e, b1, w2, b2):
    grid = _EROWS // _BR
    wspec = lambda shape: pl.BlockSpec(shape, lambda i: (0, 0))
    return pl.pallas_call(
        _edge_mlp_body,
        grid=(grid,),
        in_specs=[
            pl.BlockSpec((_BR, 128), lambda i: (i, 0)),
            pl.BlockSpec((_BR, 128), lambda i: (i + grid, 0)),
            pl.BlockSpec((_BR // 4, 128), lambda i: (i, 0)),
            wspec((128, _PK * H)), wspec((128, _PK * H)),
            wspec((_PK * FE, _PK * H)),
            wspec((1, _PK * H)), wspec((_PK * H, 128)), wspec((1, 128)),
        ],
        out_specs=pl.BlockSpec((_BR, 128), lambda i: (i, 0)),
        out_shape=jax.ShapeDtypeStruct((_EROWS, 128), jnp.float32),
    )(g_p, g_p, eap_p, w1s, w1d, w1e, b1, w2, b2)


_NP = N // _PK


def _update_body(x_ref, aggp_ref, w1x, w1a, b1, w2, b2, out_ref):
    agg = aggp_ref[0:_NP, :] + aggp_ref[_NP:2 * _NP, :]
    h = (jnp.dot(_bf(x_ref[...]), w1x[...], preferred_element_type=jnp.float32)
         + jnp.dot(_bf(agg), w1a[...], preferred_element_type=jnp.float32)
         + b1[...])
    out_ref[...] = x_ref[...] + jnp.dot(
        _bf(_silu(h)), w2[...], preferred_element_type=jnp.float32) + b2[...]


def _update(x_p, aggp_p, w1x, w1a, b1, w2, b2):
    return pl.pallas_call(
        _update_body,
        out_shape=jax.ShapeDtypeStruct((_NP, 128), jnp.float32),
    )(x_p, aggp_p, w1x, w1a, b1, w2, b2)


_FC_BK = 6400


def _fc1_body(flat_ref, w1_ref, b1_ref, out_ref):
    k = pl.program_id(0)
    part = jnp.dot(flat_ref[...], w1_ref[...], preferred_element_type=jnp.float32)

    @pl.when(k == 0)
    def _():
        out_ref[...] = part + b1_ref[...]

    @pl.when(k > 0)
    def _():
        out_ref[...] += part


def _fc1(flat, w1, b1):
    grid = (N * F) // _FC_BK
    return pl.pallas_call(
        _fc1_body,
        grid=(grid,),
        in_specs=[
            pl.BlockSpec((1, _FC_BK), lambda k: (0, k)),
            pl.BlockSpec((_FC_BK, FCH), lambda k: (k, 0)),
            pl.BlockSpec((1, FCH), lambda k: (0, 0)),
        ],
        out_specs=pl.BlockSpec((1, FCH), lambda k: (0, 0)),
        out_shape=jax.ShapeDtypeStruct((1, FCH), jnp.float32),
    )(flat, w1, b1)


def _fc_mid_body(s1_ref, ew2, eb2, dw1, db1, out_ref):
    z = jnp.dot(_silu(s1_ref[...]), ew2[...],
                preferred_element_type=jnp.float32) + eb2[...]
    out_ref[...] = _silu(
        jnp.dot(z, dw1[...], preferred_element_type=jnp.float32) + db1[...])


def _fc_mid(s1, ew2, eb2, dw1, db1):
    return pl.pallas_call(
        _fc_mid_body,
        out_shape=jax.ShapeDtypeStruct((1, FCH), jnp.float32),
    )(s1, ew2, eb2, dw1, db1)


_FC_BN = 6400


def _fc2_body(t_ref, w2_ref, b2_ref, out_ref):
    out_ref[...] = jnp.dot(t_ref[...], w2_ref[...],
                           preferred_element_type=jnp.float32) + b2_ref[...]


def _fc2(t, w2, b2):
    grid = (N * F) // _FC_BN
    return pl.pallas_call(
        _fc2_body,
        grid=(grid,),
        in_specs=[
            pl.BlockSpec((1, FCH), lambda j: (0, 0)),
            pl.BlockSpec((FCH, _FC_BN), lambda j: (0, j)),
            pl.BlockSpec((1, _FC_BN), lambda j: (0, j)),
        ],
        out_specs=pl.BlockSpec((1, _FC_BN), lambda j: (0, j)),
        out_shape=jax.ShapeDtypeStruct((1, N * F), jnp.float32),
    )(t, w2, b2)


def _kron8(w):
    return _bf(jnp.kron(jnp.eye(_PK, dtype=jnp.float32), w))


def _tile8(b):
    return jnp.tile(b, _PK).reshape(1, _PK * b.shape[0])


def _block(h_p, idx2e, dst2d, ea8, mW1, mb1, mW2, mb2, uW1, ub1, uW2, ub2):
    g = _gather(h_p.reshape(N, F), idx2e)
    msg_p = _edge_mlp(g.reshape(_GROWS, 128), ea8,
                      _kron8(mW1[:F]), _kron8(mW1[F:2 * F]),
                      _kron8(mW1[2 * F:]),
                      _tile8(mb1), _kron8(mW2), _tile8(mb2))
    aggp = _scatter(msg_p.reshape(E, F), dst2d)
    return _update(h_p, aggp.reshape(2 * _NP, 128),
                   _kron8(uW1[:F]), _kron8(uW1[F:2 * F]), _tile8(ub1),
                   _kron8(uW2), _tile8(ub2))


def kernel(x, edge_index, edge_attr, enc_mW1, enc_mb1, enc_mW2, enc_mb2,
           enc_uW1, enc_ub1, enc_uW2, enc_ub2, dec_mW1, dec_mb1, dec_mW2,
           dec_mb2, dec_uW1, dec_ub1, dec_uW2, dec_ub2, fcE_W1, fcE_b1,
           fcE_W2, fcE_b2, fcD_W1, fcD_b1, fcD_W2, fcD_b2):
    idx2e = edge_index.reshape(2 * E)
    dst2d = edge_index[1].reshape(E // _S_CH, _S_CH)
    ea8 = edge_attr.reshape(E // 32, 128)

    h_p = x.reshape(_NP, 128)
    for l in range(L):
        h_p = _block(h_p, idx2e, dst2d, ea8, enc_mW1[l], enc_mb1[l],
                     enc_mW2[l], enc_mb2[l], enc_uW1[l], enc_ub1[l],
                     enc_uW2[l], enc_ub2[l])
    s1 = _fc1(h_p.reshape(1, N * F), fcE_W1, fcE_b1.reshape(1, FCH))
    t = _fc_mid(s1, fcE_W2, fcE_b2.reshape(1, LAT), fcD_W1,
                fcD_b1.reshape(1, FCH))
    d = _fc2(t, fcD_W2, fcD_b2.reshape(1, N * F))
    h_p = d.reshape(_NP, 128)
    for l in range(L):
        h_p = _block(h_p, idx2e, dst2d, ea8, dec_mW1[l], dec_mb1[l],
                     dec_mW2[l], dec_mb2[l], dec_uW1[l], dec_ub1[l],
                     dec_uW2[l], dec_ub2[l])
    return h_p.reshape(N, F)


# speedup vs baseline: 1.3588x; 1.2211x over previous
import functools

import jax
import jax.numpy as jnp
from jax import lax
from jax.experimental import pallas as pl
from jax.experimental.pallas import tpu as pltpu
from jax.experimental.pallas import tpu_sc as plsc

N = 10000
F = 16
FE = 4
E = 320000
H = 64
LAT = 128
FCH = 256
L = 2

NC = 2
NS = 16
NW = NC * NS


_G_ROWS = 2 * E
_G_PER_W = _G_ROWS // NW
_G_CH = 128
_G_K = 2
_G_GROUP = _G_CH * _G_K
_G_NGRP = _G_PER_W // _G_GROUP
_G_TAIL = _G_PER_W - _G_NGRP * _G_GROUP


_G_STRIPE = N // NS


def _gather_body(x_hbm, idx_hbm, out_hbm, idx_v, rows_v, tail_idx_v, tail_rows_v,
                 x_shared, gsem, osem):
    sid = lax.axis_index("s")
    wid = sid * NC + lax.axis_index("c")
    base = wid * _G_PER_W
    pltpu.sync_copy(x_hbm.at[pl.ds(sid * _G_STRIPE, _G_STRIPE)],
                    x_shared.at[pl.ds(sid * _G_STRIPE, _G_STRIPE)])
    pltpu.sync_copy(idx_hbm.at[pl.ds(base, _G_PER_W)], idx_v)
    plsc.subcore_barrier()

    @pl.loop(0, _G_NGRP, step=2)
    def _group(g0):
        for b in range(2):
            g = g0 + b
            off = g * _G_GROUP
            descs = [
                pltpu.async_copy(
                    x_shared.at[idx_v.at[pl.ds(off + j * _G_CH, _G_CH)]],
                    rows_v.at[b, pl.ds(j * _G_CH, _G_CH)], gsem)
                for j in range(_G_K)
            ]
            for d in descs:
                d.wait()
            @pl.when(g > 0)
            def _():
                pltpu.make_async_copy(
                    rows_v.at[1 - b], out_hbm.at[pl.ds(0, _G_GROUP)],
                    osem).wait()
            pltpu.async_copy(rows_v.at[b],
                             out_hbm.at[pl.ds(base + off, _G_GROUP)], osem)

    toff = base + _G_NGRP * _G_GROUP
    pltpu.sync_copy(idx_hbm.at[pl.ds(toff, _G_TAIL)], tail_idx_v)
    pltpu.async_copy(x_shared.at[tail_idx_v], tail_rows_v, gsem).wait()
    pltpu.sync_copy(tail_rows_v, out_hbm.at[pl.ds(toff, _G_TAIL)])
    pltpu.make_async_copy(rows_v.at[0], out_hbm.at[pl.ds(0, _G_GROUP)],
                          osem).wait()


@functools.cache
def _sc_mesh():
    return plsc.VectorSubcoreMesh(
        core_axis_name="c", subcore_axis_name="s",
        num_cores=NC, num_subcores=NS)


_SC_PARAMS = pltpu.CompilerParams(use_tc_tiling_on_sc=False)


@functools.cache
def _gather_kernel():
    return pl.kernel(
        _gather_body,
        out_type=jax.ShapeDtypeStruct((_G_ROWS, F), jnp.float32),
        mesh=_sc_mesh(),
        compiler_params=_SC_PARAMS,
        scratch_types=[
            pltpu.VMEM((_G_PER_W,), jnp.int32),
            pltpu.VMEM((2, _G_GROUP, F), jnp.float32),
            pltpu.VMEM((_G_TAIL,), jnp.int32),
            pltpu.VMEM((_G_TAIL, F), jnp.float32),
            pltpu.VMEM_SHARED((N, F), jnp.float32),
            pltpu.SemaphoreType.DMA,
            pltpu.SemaphoreType.DMA,
        ],
    )


def _gather(x, idx):
    return _gather_kernel()(x, idx)


_S_CH = 128
_S_ROWS_PW = 78
_S_MAIN_PW = _S_ROWS_PW * _S_CH
_S_HK = _S_ROWS_PW // 2
_S_HALF = _S_HK * _S_CH
_S_TAIL = 16
_S_TAIL_BASE = NW * _S_MAIN_PW
_S_ZROWS = N // NS


def _scatter_body(msg_hbm, dst2d_hbm, out_hbm, idx2_v, msg_v,
                  tidx_v, tmsg_v, stripe_v, acc_shared, sem):
    cid = lax.axis_index("c")
    sid = lax.axis_index("s")
    wid = cid * NS + sid
    def zrow(j, _):
        stripe_v[j, :] = jnp.zeros((F,), jnp.float32)
        return _
    lax.fori_loop(0, _S_ZROWS, zrow, 0, unroll=8)
    pltpu.sync_copy(stripe_v, acc_shared.at[pl.ds(sid * _S_ZROWS, _S_ZROWS)])
    plsc.subcore_barrier()

    pltpu.sync_copy(dst2d_hbm.at[pl.ds(wid * _S_ROWS_PW, _S_ROWS_PW)], idx2_v)
    for half in range(2):
        eoff = wid * _S_MAIN_PW + half * _S_HALF
        pltpu.sync_copy(msg_hbm.at[pl.ds(eoff, _S_HALF)], msg_v)
        descs = [
            pltpu.async_copy(msg_v.at[pl.ds(k * _S_CH, _S_CH)],
                             acc_shared.at[idx2_v.at[half * _S_HK + k]],
                             sem, add=True)
            for k in range(_S_HK)
        ]
        for d in descs:
            d.wait()
    toff = _S_TAIL_BASE + wid * _S_TAIL
    pltpu.sync_copy(
        dst2d_hbm.at[_S_TAIL_BASE // _S_CH + wid // 8,
                     pl.ds((wid % 8) * _S_TAIL, _S_TAIL)], tidx_v)
    pltpu.sync_copy(msg_hbm.at[pl.ds(toff, _S_TAIL)], tmsg_v)
    pltpu.sync_copy(tmsg_v, acc_shared.at[tidx_v], add=True)
    plsc.subcore_barrier()
    pltpu.sync_copy(acc_shared.at[pl.ds(sid * _S_ZROWS, _S_ZROWS)], stripe_v)
    pltpu.sync_copy(stripe_v,
                    out_hbm.at[pl.ds(cid * N + sid * _S_ZROWS, _S_ZROWS)])


@functools.cache
def _scatter_kernel():
    return pl.kernel(
        _scatter_body,
        out_type=jax.ShapeDtypeStruct((2 * N, F), jnp.float32),
        mesh=_sc_mesh(),
        compiler_params=_SC_PARAMS,
        scratch_types=[
            pltpu.VMEM((_S_ROWS_PW, _S_CH), jnp.int32),
            pltpu.VMEM((_S_HALF, F), jnp.float32),
            pltpu.VMEM((_S_TAIL,), jnp.int32),
            pltpu.VMEM((_S_TAIL, F), jnp.float32),
            pltpu.VMEM((_S_ZROWS, F), jnp.float32),
            pltpu.VMEM_SHARED((N, F), jnp.float32),
            pltpu.SemaphoreType.DMA,
        ],
    )


def _scatter(msg, dst2d):
    return _scatter_kernel()(msg, dst2d)


_PK = 8
_BR = 800
_EROWS = E // _PK
_GROWS = _G_ROWS // _PK


def _silu(v):
    return v * jax.nn.sigmoid(v)


def _bf(v):
    return v.astype(jnp.bfloat16)


def _edge_mlp_body(xs_ref, xd_ref, ea_ref, w1s, w1d, w1e, b1, w2, b2, out_ref):
    ea32 = ea_ref[...]
    ea8 = jnp.stack(
        [ea32[:, 32 * t:32 * (t + 1)] for t in range(4)],
        axis=1).reshape(_BR, _PK * FE)
    h = (jnp.dot(_bf(xs_ref[...]), w1s[...], preferred_element_type=jnp.float32)
         + jnp.dot(_bf(xd_ref[...]), w1d[...], preferred_element_type=jnp.float32)
         + jnp.dot(_bf(ea8), w1e[...], preferred_element_type=jnp.float32)
         + b1[...])
    out_ref[...] = jnp.dot(_bf(_silu(h)), w2[...],
                           preferred_element_type=jnp.float32) + b2[...]


def _edge_mlp(g_p, eap_p, w1s, w1d, w1e, b1, w2, b2):
    grid = _EROWS // _BR
    wspec = lambda shape: pl.BlockSpec(shape, lambda i: (0, 0))
    return pl.pallas_call(
        _edge_mlp_body,
        grid=(grid,),
        in_specs=[
            pl.BlockSpec((_BR, 128), lambda i: (i, 0)),
            pl.BlockSpec((_BR, 128), lambda i: (i + grid, 0)),
            pl.BlockSpec((_BR // 4, 128), lambda i: (i, 0)),
            wspec((128, _PK * H)), wspec((128, _PK * H)),
            wspec((_PK * FE, _PK * H)),
            wspec((1, _PK * H)), wspec((_PK * H, 128)), wspec((1, 128)),
        ],
        out_specs=pl.BlockSpec((_BR, 128), lambda i: (i, 0)),
        out_shape=jax.ShapeDtypeStruct((_EROWS, 128), jnp.float32),
    )(g_p, g_p, eap_p, w1s, w1d, w1e, b1, w2, b2)


_NP = N // _PK


def _update_body(x_ref, aggp_ref, w1x, w1a, b1, w2, b2, out_ref):
    agg = aggp_ref[0:_NP, :] + aggp_ref[_NP:2 * _NP, :]
    h = (jnp.dot(_bf(x_ref[...]), w1x[...], preferred_element_type=jnp.float32)
         + jnp.dot(_bf(agg), w1a[...], preferred_element_type=jnp.float32)
         + b1[...])
    out_ref[...] = x_ref[...] + jnp.dot(
        _bf(_silu(h)), w2[...], preferred_element_type=jnp.float32) + b2[...]


def _update(x_p, aggp_p, w1x, w1a, b1, w2, b2):
    return pl.pallas_call(
        _update_body,
        out_shape=jax.ShapeDtypeStruct((_NP, 128), jnp.float32),
    )(x_p, aggp_p, w1x, w1a, b1, w2, b2)


_FC_BK = 6400


def _fc1_body(flat_ref, w1_ref, b1_ref, out_ref):
    k = pl.program_id(0)
    part = jnp.dot(flat_ref[...], w1_ref[...], preferred_element_type=jnp.float32)

    @pl.when(k == 0)
    def _():
        out_ref[...] = part + b1_ref[...]

    @pl.when(k > 0)
    def _():
        out_ref[...] += part


def _fc1(flat, w1, b1):
    grid = (N * F) // _FC_BK
    return pl.pallas_call(
        _fc1_body,
        grid=(grid,),
        in_specs=[
            pl.BlockSpec((1, _FC_BK), lambda k: (0, k)),
            pl.BlockSpec((_FC_BK, FCH), lambda k: (k, 0)),
            pl.BlockSpec((1, FCH), lambda k: (0, 0)),
        ],
        out_specs=pl.BlockSpec((1, FCH), lambda k: (0, 0)),
        out_shape=jax.ShapeDtypeStruct((1, FCH), jnp.float32),
    )(flat, w1, b1)


def _fc_mid_body(s1_ref, ew2, eb2, dw1, db1, out_ref):
    z = jnp.dot(_silu(s1_ref[...]), ew2[...],
                preferred_element_type=jnp.float32) + eb2[...]
    out_ref[...] = _silu(
        jnp.dot(z, dw1[...], preferred_element_type=jnp.float32) + db1[...])


def _fc_mid(s1, ew2, eb2, dw1, db1):
    return pl.pallas_call(
        _fc_mid_body,
        out_shape=jax.ShapeDtypeStruct((1, FCH), jnp.float32),
    )(s1, ew2, eb2, dw1, db1)


_FC_BN = 6400


def _fc2_body(t_ref, w2_ref, b2_ref, out_ref):
    out_ref[...] = jnp.dot(t_ref[...], w2_ref[...],
                           preferred_element_type=jnp.float32) + b2_ref[...]


def _fc2(t, w2, b2):
    grid = (N * F) // _FC_BN
    return pl.pallas_call(
        _fc2_body,
        grid=(grid,),
        in_specs=[
            pl.BlockSpec((1, FCH), lambda j: (0, 0)),
            pl.BlockSpec((FCH, _FC_BN), lambda j: (0, j)),
            pl.BlockSpec((1, _FC_BN), lambda j: (0, j)),
        ],
        out_specs=pl.BlockSpec((1, _FC_BN), lambda j: (0, j)),
        out_shape=jax.ShapeDtypeStruct((1, N * F), jnp.float32),
    )(t, w2, b2)


def _kron8(w):
    return _bf(jnp.kron(jnp.eye(_PK, dtype=jnp.float32), w))


def _tile8(b):
    return jnp.tile(b, _PK).reshape(1, _PK * b.shape[0])


def _block(h_p, idx2e, dst2d, ea8, mW1, mb1, mW2, mb2, uW1, ub1, uW2, ub2):
    g = _gather(h_p.reshape(N, F), idx2e)
    msg_p = _edge_mlp(g.reshape(_GROWS, 128), ea8,
                      _kron8(mW1[:F]), _kron8(mW1[F:2 * F]),
                      _kron8(mW1[2 * F:]),
                      _tile8(mb1), _kron8(mW2), _tile8(mb2))
    aggp = _scatter(msg_p.reshape(E, F), dst2d)
    return _update(h_p, aggp.reshape(2 * _NP, 128),
                   _kron8(uW1[:F]), _kron8(uW1[F:2 * F]), _tile8(ub1),
                   _kron8(uW2), _tile8(ub2))


def kernel(x, edge_index, edge_attr, enc_mW1, enc_mb1, enc_mW2, enc_mb2,
           enc_uW1, enc_ub1, enc_uW2, enc_ub2, dec_mW1, dec_mb1, dec_mW2,
           dec_mb2, dec_uW1, dec_ub1, dec_uW2, dec_ub2, fcE_W1, fcE_b1,
           fcE_W2, fcE_b2, fcD_W1, fcD_b1, fcD_W2, fcD_b2):
    idx2e = edge_index.reshape(2 * E)
    dst2d = edge_index[1].reshape(E // _S_CH, _S_CH)
    ea8 = edge_attr.T.reshape(4, E // 32, 32).transpose(1, 2, 0).reshape(
        E // 32, 128)

    h_p = x.reshape(_NP, 128)
    for l in range(L):
        h_p = _block(h_p, idx2e, dst2d, ea8, enc_mW1[l], enc_mb1[l],
                     enc_mW2[l], enc_mb2[l], enc_uW1[l], enc_ub1[l],
                     enc_uW2[l], enc_ub2[l])
    s1 = _fc1(h_p.reshape(1, N * F), fcE_W1, fcE_b1.reshape(1, FCH))
    t = _fc_mid(s1, fcE_W2, fcE_b2.reshape(1, LAT), fcD_W1,
                fcD_b1.reshape(1, FCH))
    d = _fc2(t, fcD_W2, fcD_b2.reshape(1, N * F))
    h_p = d.reshape(_NP, 128)
    for l in range(L):
        h_p = _block(h_p, idx2e, dst2d, ea8, dec_mW1[l], dec_mb1[l],
                     dec_mW2[l], dec_mb2[l], dec_uW1[l], dec_ub1[l],
                     dec_uW2[l], dec_ub2[l])
    return h_p.reshape(N, F)


# speedup vs baseline: 1.4251x; 1.0487x over previous
import functools

import jax
import jax.numpy as jnp
from jax import lax
from jax.experimental import pallas as pl
from jax.experimental.pallas import tpu as pltpu
from jax.experimental.pallas import tpu_sc as plsc

N = 10000
F = 16
FE = 4
E = 320000
H = 64
LAT = 128
FCH = 256
L = 2

NC = 2
NS = 16
NW = NC * NS


_G_ROWS = 2 * E
_G_PER_W = _G_ROWS // NW
_G_CH = 128
_G_K = 2
_G_GROUP = _G_CH * _G_K
_G_NGRP = _G_PER_W // _G_GROUP
_G_TAIL = _G_PER_W - _G_NGRP * _G_GROUP


_G_STRIPE = N // NS


def _gather_body(x_hbm, idx_hbm, out_hbm, idx_v, rows_v, tail_idx_v, tail_rows_v,
                 x_shared, gsem, osem):
    sid = lax.axis_index("s")
    wid = sid * NC + lax.axis_index("c")
    base = wid * _G_PER_W
    pltpu.sync_copy(x_hbm.at[pl.ds(sid * _G_STRIPE, _G_STRIPE)],
                    x_shared.at[pl.ds(sid * _G_STRIPE, _G_STRIPE)])
    pltpu.sync_copy(idx_hbm.at[pl.ds(base, _G_PER_W)], idx_v)
    plsc.subcore_barrier()

    @pl.loop(0, _G_NGRP, step=2)
    def _group(g0):
        for b in range(2):
            g = g0 + b
            off = g * _G_GROUP
            descs = [
                pltpu.async_copy(
                    x_shared.at[idx_v.at[pl.ds(off + j * _G_CH, _G_CH)]],
                    rows_v.at[b, pl.ds(j * _G_CH, _G_CH)], gsem)
                for j in range(_G_K)
            ]
            for d in descs:
                d.wait()
            @pl.when(g > 0)
            def _():
                pltpu.make_async_copy(
                    rows_v.at[1 - b], out_hbm.at[pl.ds(0, _G_GROUP)],
                    osem).wait()
            pltpu.async_copy(rows_v.at[b],
                             out_hbm.at[pl.ds(base + off, _G_GROUP)], osem)

    toff = base + _G_NGRP * _G_GROUP
    pltpu.sync_copy(idx_hbm.at[pl.ds(toff, _G_TAIL)], tail_idx_v)
    pltpu.async_copy(x_shared.at[tail_idx_v], tail_rows_v, gsem).wait()
    pltpu.sync_copy(tail_rows_v, out_hbm.at[pl.ds(toff, _G_TAIL)])
    pltpu.make_async_copy(rows_v.at[0], out_hbm.at[pl.ds(0, _G_GROUP)],
                          osem).wait()


@functools.cache
def _sc_mesh():
    return plsc.VectorSubcoreMesh(
        core_axis_name="c", subcore_axis_name="s",
        num_cores=NC, num_subcores=NS)


_SC_PARAMS = pltpu.CompilerParams(use_tc_tiling_on_sc=False)


@functools.cache
def _gather_kernel():
    return pl.kernel(
        _gather_body,
        out_type=jax.ShapeDtypeStruct((_G_ROWS, F), jnp.float32),
        mesh=_sc_mesh(),
        compiler_params=_SC_PARAMS,
        scratch_types=[
            pltpu.VMEM((_G_PER_W,), jnp.int32),
            pltpu.VMEM((2, _G_GROUP, F), jnp.float32),
            pltpu.VMEM((_G_TAIL,), jnp.int32),
            pltpu.VMEM((_G_TAIL, F), jnp.float32),
            pltpu.VMEM_SHARED((N, F), jnp.float32),
            pltpu.SemaphoreType.DMA,
            pltpu.SemaphoreType.DMA,
        ],
    )


def _gather(x, idx):
    return _gather_kernel()(x, idx)


_S_CH = 128
_S_ROWS_PW = 78
_S_MAIN_PW = _S_ROWS_PW * _S_CH
_S_HK = _S_ROWS_PW // 2
_S_HALF = _S_HK * _S_CH
_S_TAIL = 16
_S_TAIL_BASE = NW * _S_MAIN_PW
_S_ZROWS = N // NS


def _scatter_body(msg_hbm, dst2d_hbm, out_hbm, idx2_v, msg_v,
                  tidx_v, tmsg_v, stripe_v, acc_shared, sem):
    cid = lax.axis_index("c")
    sid = lax.axis_index("s")
    wid = cid * NS + sid
    def zrow(j, _):
        stripe_v[j, :] = jnp.zeros((F,), jnp.float32)
        return _
    lax.fori_loop(0, _S_ZROWS, zrow, 0, unroll=8)
    pltpu.sync_copy(stripe_v, acc_shared.at[pl.ds(sid * _S_ZROWS, _S_ZROWS)])
    plsc.subcore_barrier()

    pltpu.sync_copy(dst2d_hbm.at[pl.ds(wid * _S_ROWS_PW, _S_ROWS_PW)], idx2_v)
    for half in range(2):
        eoff = wid * _S_MAIN_PW + half * _S_HALF
        pltpu.sync_copy(msg_hbm.at[pl.ds(eoff, _S_HALF)], msg_v)
        descs = [
            pltpu.async_copy(msg_v.at[pl.ds(k * _S_CH, _S_CH)],
                             acc_shared.at[idx2_v.at[half * _S_HK + k]],
                             sem, add=True)
            for k in range(_S_HK)
        ]
        for d in descs:
            d.wait()
    toff = _S_TAIL_BASE + wid * _S_TAIL
    pltpu.sync_copy(
        dst2d_hbm.at[_S_TAIL_BASE // _S_CH + wid // 8,
                     pl.ds((wid % 8) * _S_TAIL, _S_TAIL)], tidx_v)
    pltpu.sync_copy(msg_hbm.at[pl.ds(toff, _S_TAIL)], tmsg_v)
    pltpu.sync_copy(tmsg_v, acc_shared.at[tidx_v], add=True)
    plsc.subcore_barrier()
    pltpu.sync_copy(acc_shared.at[pl.ds(sid * _S_ZROWS, _S_ZROWS)], stripe_v)
    pltpu.sync_copy(stripe_v,
                    out_hbm.at[pl.ds(cid * N + sid * _S_ZROWS, _S_ZROWS)])


@functools.cache
def _scatter_kernel():
    return pl.kernel(
        _scatter_body,
        out_type=jax.ShapeDtypeStruct((2 * N, F), jnp.float32),
        mesh=_sc_mesh(),
        compiler_params=_SC_PARAMS,
        scratch_types=[
            pltpu.VMEM((_S_ROWS_PW, _S_CH), jnp.int32),
            pltpu.VMEM((_S_HALF, F), jnp.float32),
            pltpu.VMEM((_S_TAIL,), jnp.int32),
            pltpu.VMEM((_S_TAIL, F), jnp.float32),
            pltpu.VMEM((_S_ZROWS, F), jnp.float32),
            pltpu.VMEM_SHARED((N, F), jnp.float32),
            pltpu.SemaphoreType.DMA,
        ],
    )


def _scatter(msg, dst2d):
    return _scatter_kernel()(msg, dst2d)


_PK = 8
_BR = 1600
_EROWS = E // _PK
_GROWS = _G_ROWS // _PK


def _silu(v):
    return v * jax.nn.sigmoid(v)


def _bf(v):
    return v.astype(jnp.bfloat16)


def _edge_mlp_body(xs_ref, xd_ref, ea_ref, w1s, w1d, w1e, b1, w2, b2, out_ref):
    ea32 = ea_ref[...]
    ea8 = jnp.stack(
        [ea32[:, 32 * t:32 * (t + 1)] for t in range(4)],
        axis=1).reshape(_BR, _PK * FE)
    h = (jnp.dot(_bf(xs_ref[...]), w1s[...], preferred_element_type=jnp.float32)
         + jnp.dot(_bf(xd_ref[...]), w1d[...], preferred_element_type=jnp.float32)
         + jnp.dot(_bf(ea8), w1e[...], preferred_element_type=jnp.float32)
         + b1[...])
    out_ref[...] = jnp.dot(_bf(_silu(h)), w2[...],
                           preferred_element_type=jnp.float32) + b2[...]


def _edge_mlp(g_p, eap_p, w1s, w1d, w1e, b1, w2, b2):
    grid = _EROWS // _BR
    wspec = lambda shape: pl.BlockSpec(shape, lambda i: (0, 0))
    return pl.pallas_call(
        _edge_mlp_body,
        grid=(grid,),
        in_specs=[
            pl.BlockSpec((_BR, 128), lambda i: (i, 0)),
            pl.BlockSpec((_BR, 128), lambda i: (i + grid, 0)),
            pl.BlockSpec((_BR // 4, 128), lambda i: (i, 0)),
            wspec((128, _PK * H)), wspec((128, _PK * H)),
            wspec((_PK * FE, _PK * H)),
            wspec((1, _PK * H)), wspec((_PK * H, 128)), wspec((1, 128)),
        ],
        out_specs=pl.BlockSpec((_BR, 128), lambda i: (i, 0)),
        out_shape=jax.ShapeDtypeStruct((_EROWS, 128), jnp.float32),
    )(g_p, g_p, eap_p, w1s, w1d, w1e, b1, w2, b2)


_NP = N // _PK


def _update_body(x_ref, aggp_ref, w1x, w1a, b1, w2, b2, out_ref):
    agg = aggp_ref[0:_NP, :] + aggp_ref[_NP:2 * _NP, :]
    h = (jnp.dot(_bf(x_ref[...]), w1x[...], preferred_element_type=jnp.float32)
         + jnp.dot(_bf(agg), w1a[...], preferred_element_type=jnp.float32)
         + b1[...])
    out_ref[...] = x_ref[...] + jnp.dot(
        _bf(_silu(h)), w2[...], preferred_element_type=jnp.float32) + b2[...]


def _update(x_p, aggp_p, w1x, w1a, b1, w2, b2):
    return pl.pallas_call(
        _update_body,
        out_shape=jax.ShapeDtypeStruct((_NP, 128), jnp.float32),
    )(x_p, aggp_p, w1x, w1a, b1, w2, b2)


_FC_BK = 6400


def _fc1_body(flat_ref, w1_ref, b1_ref, out_ref):
    k = pl.program_id(0)
    part = jnp.dot(flat_ref[...], w1_ref[...], preferred_element_type=jnp.float32)

    @pl.when(k == 0)
    def _():
        out_ref[...] = part + b1_ref[...]

    @pl.when(k > 0)
    def _():
        out_ref[...] += part


def _fc1(flat, w1, b1):
    grid = (N * F) // _FC_BK
    return pl.pallas_call(
        _fc1_body,
        grid=(grid,),
        in_specs=[
            pl.BlockSpec((1, _FC_BK), lambda k: (0, k)),
            pl.BlockSpec((_FC_BK, FCH), lambda k: (k, 0)),
            pl.BlockSpec((1, FCH), lambda k: (0, 0)),
        ],
        out_specs=pl.BlockSpec((1, FCH), lambda k: (0, 0)),
        out_shape=jax.ShapeDtypeStruct((1, FCH), jnp.float32),
    )(flat, w1, b1)


def _fc_mid_body(s1_ref, ew2, eb2, dw1, db1, out_ref):
    z = jnp.dot(_silu(s1_ref[...]), ew2[...],
                preferred_element_type=jnp.float32) + eb2[...]
    out_ref[...] = _silu(
        jnp.dot(z, dw1[...], preferred_element_type=jnp.float32) + db1[...])


def _fc_mid(s1, ew2, eb2, dw1, db1):
    return pl.pallas_call(
        _fc_mid_body,
        out_shape=jax.ShapeDtypeStruct((1, FCH), jnp.float32),
    )(s1, ew2, eb2, dw1, db1)


_FC_BN = 6400


def _fc2_body(t_ref, w2_ref, b2_ref, out_ref):
    out_ref[...] = jnp.dot(t_ref[...], w2_ref[...],
                           preferred_element_type=jnp.float32) + b2_ref[...]


def _fc2(t, w2, b2):
    grid = (N * F) // _FC_BN
    return pl.pallas_call(
        _fc2_body,
        grid=(grid,),
        in_specs=[
            pl.BlockSpec((1, FCH), lambda j: (0, 0)),
            pl.BlockSpec((FCH, _FC_BN), lambda j: (0, j)),
            pl.BlockSpec((1, _FC_BN), lambda j: (0, j)),
        ],
        out_specs=pl.BlockSpec((1, _FC_BN), lambda j: (0, j)),
        out_shape=jax.ShapeDtypeStruct((1, N * F), jnp.float32),
    )(t, w2, b2)


def _kron8(w):
    return _bf(jnp.kron(jnp.eye(_PK, dtype=jnp.float32), w))


def _tile8(b):
    return jnp.tile(b, _PK).reshape(1, _PK * b.shape[0])


def _block(h_p, idx2e, dst2d, ea8, mW1, mb1, mW2, mb2, uW1, ub1, uW2, ub2):
    g = _gather(h_p.reshape(N, F), idx2e)
    msg_p = _edge_mlp(g.reshape(_GROWS, 128), ea8,
                      _kron8(mW1[:F]), _kron8(mW1[F:2 * F]),
                      _kron8(mW1[2 * F:]),
                      _tile8(mb1), _kron8(mW2), _tile8(mb2))
    aggp = _scatter(msg_p.reshape(E, F), dst2d)
    return _update(h_p, aggp.reshape(2 * _NP, 128),
                   _kron8(uW1[:F]), _kron8(uW1[F:2 * F]), _tile8(ub1),
                   _kron8(uW2), _tile8(ub2))


def kernel(x, edge_index, edge_attr, enc_mW1, enc_mb1, enc_mW2, enc_mb2,
           enc_uW1, enc_ub1, enc_uW2, enc_ub2, dec_mW1, dec_mb1, dec_mW2,
           dec_mb2, dec_uW1, dec_ub1, dec_uW2, dec_ub2, fcE_W1, fcE_b1,
           fcE_W2, fcE_b2, fcD_W1, fcD_b1, fcD_W2, fcD_b2):
    idx2e = edge_index.reshape(2 * E)
    dst2d = edge_index[1].reshape(E // _S_CH, _S_CH)
    ea8 = edge_attr.T.reshape(4, E // 32, 32).transpose(1, 2, 0).reshape(
        E // 32, 128)

    h_p = x.reshape(_NP, 128)
    for l in range(L):
        h_p = _block(h_p, idx2e, dst2d, ea8, enc_mW1[l], enc_mb1[l],
                     enc_mW2[l], enc_mb2[l], enc_uW1[l], enc_ub1[l],
                     enc_uW2[l], enc_ub2[l])
    s1 = _fc1(h_p.reshape(1, N * F), fcE_W1, fcE_b1.reshape(1, FCH))
    t = _fc_mid(s1, fcE_W2, fcE_b2.reshape(1, LAT), fcD_W1,
                fcD_b1.reshape(1, FCH))
    d = _fc2(t, fcD_W2, fcD_b2.reshape(1, N * F))
    h_p = d.reshape(_NP, 128)
    for l in range(L):
        h_p = _block(h_p, idx2e, dst2d, ea8, dec_mW1[l], dec_mb1[l],
                     dec_mW2[l], dec_mb2[l], dec_uW1[l], dec_ub1[l],
                     dec_uW2[l], dec_ub2[l])
    return h_p.reshape(N, F)


# speedup vs baseline: 1.4329x; 1.0055x over previous
import functools

import jax
import jax.numpy as jnp
from jax import lax
from jax.experimental import pallas as pl
from jax.experimental.pallas import tpu as pltpu
from jax.experimental.pallas import tpu_sc as plsc

N = 10000
F = 16
FE = 4
E = 320000
H = 64
LAT = 128
FCH = 256
L = 2

NC = 2
NS = 16
NW = NC * NS


_G_ROWS = 2 * E
_G_PER_W = _G_ROWS // NW
_G_CH = 128
_G_K = 2
_G_GROUP = _G_CH * _G_K
_G_NGRP = _G_PER_W // _G_GROUP
_G_TAIL = _G_PER_W - _G_NGRP * _G_GROUP


_G_STRIPE = N // NS


def _gather_body(x_hbm, idx_hbm, out_hbm, idx_v, rows_v, tail_idx_v, tail_rows_v,
                 x_shared, gsem, osem):
    sid = lax.axis_index("s")
    wid = sid * NC + lax.axis_index("c")
    base = wid * _G_PER_W
    pltpu.sync_copy(x_hbm.at[pl.ds(sid * _G_STRIPE, _G_STRIPE)],
                    x_shared.at[pl.ds(sid * _G_STRIPE, _G_STRIPE)])
    pltpu.sync_copy(idx_hbm.at[pl.ds(base, _G_PER_W)], idx_v)
    plsc.subcore_barrier()

    @pl.loop(0, _G_NGRP, step=2)
    def _group(g0):
        for b in range(2):
            g = g0 + b
            off = g * _G_GROUP
            descs = [
                pltpu.async_copy(
                    x_shared.at[idx_v.at[pl.ds(off + j * _G_CH, _G_CH)]],
                    rows_v.at[b, pl.ds(j * _G_CH, _G_CH)], gsem)
                for j in range(_G_K)
            ]
            for d in descs:
                d.wait()
            @pl.when(g > 0)
            def _():
                pltpu.make_async_copy(
                    rows_v.at[1 - b], out_hbm.at[pl.ds(0, _G_GROUP)],
                    osem).wait()
            pltpu.async_copy(rows_v.at[b],
                             out_hbm.at[pl.ds(base + off, _G_GROUP)], osem)

    toff = base + _G_NGRP * _G_GROUP
    pltpu.sync_copy(idx_hbm.at[pl.ds(toff, _G_TAIL)], tail_idx_v)
    pltpu.async_copy(x_shared.at[tail_idx_v], tail_rows_v, gsem).wait()
    pltpu.sync_copy(tail_rows_v, out_hbm.at[pl.ds(toff, _G_TAIL)])
    pltpu.make_async_copy(rows_v.at[0], out_hbm.at[pl.ds(0, _G_GROUP)],
                          osem).wait()


@functools.cache
def _sc_mesh():
    return plsc.VectorSubcoreMesh(
        core_axis_name="c", subcore_axis_name="s",
        num_cores=NC, num_subcores=NS)


_SC_PARAMS = pltpu.CompilerParams(use_tc_tiling_on_sc=False)


@functools.cache
def _gather_kernel():
    return pl.kernel(
        _gather_body,
        out_type=jax.ShapeDtypeStruct((_G_ROWS, F), jnp.float32),
        mesh=_sc_mesh(),
        compiler_params=_SC_PARAMS,
        scratch_types=[
            pltpu.VMEM((_G_PER_W,), jnp.int32),
            pltpu.VMEM((2, _G_GROUP, F), jnp.float32),
            pltpu.VMEM((_G_TAIL,), jnp.int32),
            pltpu.VMEM((_G_TAIL, F), jnp.float32),
            pltpu.VMEM_SHARED((N, F), jnp.float32),
            pltpu.SemaphoreType.DMA,
            pltpu.SemaphoreType.DMA,
        ],
    )


def _gather(x, idx):
    return _gather_kernel()(x, idx)


_S_CH = 128
_S_ROWS_PW = 78
_S_MAIN_PW = _S_ROWS_PW * _S_CH
_S_HK = _S_ROWS_PW // 2
_S_HALF = _S_HK * _S_CH
_S_TAIL = 16
_S_TAIL_BASE = NW * _S_MAIN_PW
_S_ZROWS = N // NS


def _scatter_body(msg_hbm, dst2d_hbm, out_hbm, idx2_v, msg_v,
                  tidx_v, tmsg_v, stripe_v, acc_shared, sem):
    cid = lax.axis_index("c")
    sid = lax.axis_index("s")
    wid = cid * NS + sid
    def zrow(j, _):
        stripe_v[j, :] = jnp.zeros((F,), jnp.float32)
        return _
    lax.fori_loop(0, _S_ZROWS, zrow, 0, unroll=8)
    pltpu.sync_copy(stripe_v, acc_shared.at[pl.ds(sid * _S_ZROWS, _S_ZROWS)])
    plsc.subcore_barrier()

    pltpu.sync_copy(dst2d_hbm.at[pl.ds(wid * _S_ROWS_PW, _S_ROWS_PW)], idx2_v)
    for half in range(2):
        eoff = wid * _S_MAIN_PW + half * _S_HALF
        pltpu.sync_copy(msg_hbm.at[pl.ds(eoff, _S_HALF)], msg_v)
        descs = [
            pltpu.async_copy(msg_v.at[pl.ds(k * _S_CH, _S_CH)],
                             acc_shared.at[idx2_v.at[half * _S_HK + k]],
                             sem, add=True)
            for k in range(_S_HK)
        ]
        for d in descs:
            d.wait()
    toff = _S_TAIL_BASE + wid * _S_TAIL
    pltpu.sync_copy(
        dst2d_hbm.at[_S_TAIL_BASE // _S_CH + wid // 8,
                     pl.ds((wid % 8) * _S_TAIL, _S_TAIL)], tidx_v)
    pltpu.sync_copy(msg_hbm.at[pl.ds(toff, _S_TAIL)], tmsg_v)
    pltpu.sync_copy(tmsg_v, acc_shared.at[tidx_v], add=True)
    plsc.subcore_barrier()
    pltpu.sync_copy(acc_shared.at[pl.ds(sid * _S_ZROWS, _S_ZROWS)], stripe_v)
    pltpu.sync_copy(stripe_v,
                    out_hbm.at[pl.ds(cid * N + sid * _S_ZROWS, _S_ZROWS)])


@functools.cache
def _scatter_kernel():
    return pl.kernel(
        _scatter_body,
        out_type=jax.ShapeDtypeStruct((2 * N, F), jnp.float32),
        mesh=_sc_mesh(),
        compiler_params=_SC_PARAMS,
        scratch_types=[
            pltpu.VMEM((_S_ROWS_PW, _S_CH), jnp.int32),
            pltpu.VMEM((_S_HALF, F), jnp.float32),
            pltpu.VMEM((_S_TAIL,), jnp.int32),
            pltpu.VMEM((_S_TAIL, F), jnp.float32),
            pltpu.VMEM((_S_ZROWS, F), jnp.float32),
            pltpu.VMEM_SHARED((N, F), jnp.float32),
            pltpu.SemaphoreType.DMA,
        ],
    )


def _scatter(msg, dst2d):
    return _scatter_kernel()(msg, dst2d)


_PK = 8
_BR = 1600
_EROWS = E // _PK
_GROWS = _G_ROWS // _PK


def _silu(v):
    return v * jax.nn.sigmoid(v)


def _bf(v):
    return v.astype(jnp.bfloat16)


def _edge_mlp_body(xs_ref, xd_ref, ea_ref, w1s, w1d, w1e, b1, w2, b2, out_ref):
    ea32 = ea_ref[...]
    ea8 = jnp.stack(
        [ea32[:, 32 * t:32 * (t + 1)] for t in range(4)],
        axis=1).reshape(_BR, _PK * FE)
    h = (jnp.dot(_bf(xs_ref[...]), w1s[...], preferred_element_type=jnp.float32)
         + jnp.dot(_bf(xd_ref[...]), w1d[...], preferred_element_type=jnp.float32)
         + jnp.dot(_bf(ea8), w1e[...], preferred_element_type=jnp.float32)
         + b1[...])
    out_ref[...] = jnp.dot(_silu(_bf(h)), w2[...],
                           preferred_element_type=jnp.float32) + b2[...]


def _edge_mlp(g_p, eap_p, w1s, w1d, w1e, b1, w2, b2):
    grid = _EROWS // _BR
    wspec = lambda shape: pl.BlockSpec(shape, lambda i: (0, 0))
    return pl.pallas_call(
        _edge_mlp_body,
        grid=(grid,),
        in_specs=[
            pl.BlockSpec((_BR, 128), lambda i: (i, 0)),
            pl.BlockSpec((_BR, 128), lambda i: (i + grid, 0)),
            pl.BlockSpec((_BR // 4, 128), lambda i: (i, 0)),
            wspec((128, _PK * H)), wspec((128, _PK * H)),
            wspec((_PK * FE, _PK * H)),
            wspec((1, _PK * H)), wspec((_PK * H, 128)), wspec((1, 128)),
        ],
        out_specs=pl.BlockSpec((_BR, 128), lambda i: (i, 0)),
        out_shape=jax.ShapeDtypeStruct((_EROWS, 128), jnp.float32),
    )(g_p, g_p, eap_p, w1s, w1d, w1e, b1, w2, b2)


_NP = N // _PK


def _update_body(x_ref, aggp_ref, w1x, w1a, b1, w2, b2, out_ref):
    agg = aggp_ref[0:_NP, :] + aggp_ref[_NP:2 * _NP, :]
    h = (jnp.dot(_bf(x_ref[...]), w1x[...], preferred_element_type=jnp.float32)
         + jnp.dot(_bf(agg), w1a[...], preferred_element_type=jnp.float32)
         + b1[...])
    out_ref[...] = x_ref[...] + jnp.dot(
        _bf(_silu(h)), w2[...], preferred_element_type=jnp.float32) + b2[...]


def _update(x_p, aggp_p, w1x, w1a, b1, w2, b2):
    return pl.pallas_call(
        _update_body,
        out_shape=jax.ShapeDtypeStruct((_NP, 128), jnp.float32),
    )(x_p, aggp_p, w1x, w1a, b1, w2, b2)


_FC_BK = 6400


def _fc1_body(flat_ref, w1_ref, b1_ref, out_ref):
    k = pl.program_id(0)
    part = jnp.dot(flat_ref[...], w1_ref[...], preferred_element_type=jnp.float32)

    @pl.when(k == 0)
    def _():
        out_ref[...] = part + b1_ref[...]

    @pl.when(k > 0)
    def _():
        out_ref[...] += part


def _fc1(flat, w1, b1):
    grid = (N * F) // _FC_BK
    return pl.pallas_call(
        _fc1_body,
        grid=(grid,),
        in_specs=[
            pl.BlockSpec((1, _FC_BK), lambda k: (0, k)),
            pl.BlockSpec((_FC_BK, FCH), lambda k: (k, 0)),
            pl.BlockSpec((1, FCH), lambda k: (0, 0)),
        ],
        out_specs=pl.BlockSpec((1, FCH), lambda k: (0, 0)),
        out_shape=jax.ShapeDtypeStruct((1, FCH), jnp.float32),
    )(flat, w1, b1)


def _fc_mid_body(s1_ref, ew2, eb2, dw1, db1, out_ref):
    z = jnp.dot(_silu(s1_ref[...]), ew2[...],
                preferred_element_type=jnp.float32) + eb2[...]
    out_ref[...] = _silu(
        jnp.dot(z, dw1[...], preferred_element_type=jnp.float32) + db1[...])


def _fc_mid(s1, ew2, eb2, dw1, db1):
    return pl.pallas_call(
        _fc_mid_body,
        out_shape=jax.ShapeDtypeStruct((1, FCH), jnp.float32),
    )(s1, ew2, eb2, dw1, db1)


_FC_BN = 6400


def _fc2_body(t_ref, w2_ref, b2_ref, out_ref):
    out_ref[...] = jnp.dot(t_ref[...], w2_ref[...],
                           preferred_element_type=jnp.float32) + b2_ref[...]


def _fc2(t, w2, b2):
    grid = (N * F) // _FC_BN
    return pl.pallas_call(
        _fc2_body,
        grid=(grid,),
        in_specs=[
            pl.BlockSpec((1, FCH), lambda j: (0, 0)),
            pl.BlockSpec((FCH, _FC_BN), lambda j: (0, j)),
            pl.BlockSpec((1, _FC_BN), lambda j: (0, j)),
        ],
        out_specs=pl.BlockSpec((1, _FC_BN), lambda j: (0, j)),
        out_shape=jax.ShapeDtypeStruct((1, N * F), jnp.float32),
    )(t, w2, b2)


def _kron8(w):
    return _bf(jnp.kron(jnp.eye(_PK, dtype=jnp.float32), w))


def _tile8(b):
    return jnp.tile(b, _PK).reshape(1, _PK * b.shape[0])


def _block(h_p, idx2e, dst2d, ea8, mW1, mb1, mW2, mb2, uW1, ub1, uW2, ub2):
    g = _gather(h_p.reshape(N, F), idx2e)
    msg_p = _edge_mlp(g.reshape(_GROWS, 128), ea8,
                      _kron8(mW1[:F]), _kron8(mW1[F:2 * F]),
                      _kron8(mW1[2 * F:]),
                      _tile8(mb1), _kron8(mW2), _tile8(mb2))
    aggp = _scatter(msg_p.reshape(E, F), dst2d)
    return _update(h_p, aggp.reshape(2 * _NP, 128),
                   _kron8(uW1[:F]), _kron8(uW1[F:2 * F]), _tile8(ub1),
                   _kron8(uW2), _tile8(ub2))


def kernel(x, edge_index, edge_attr, enc_mW1, enc_mb1, enc_mW2, enc_mb2,
           enc_uW1, enc_ub1, enc_uW2, enc_ub2, dec_mW1, dec_mb1, dec_mW2,
           dec_mb2, dec_uW1, dec_ub1, dec_uW2, dec_ub2, fcE_W1, fcE_b1,
           fcE_W2, fcE_b2, fcD_W1, fcD_b1, fcD_W2, fcD_b2):
    idx2e = edge_index.reshape(2 * E)
    dst2d = edge_index[1].reshape(E // _S_CH, _S_CH)
    ea8 = edge_attr.T.reshape(4, E // 32, 32).transpose(1, 2, 0).reshape(
        E // 32, 128)

    h_p = x.reshape(_NP, 128)
    for l in range(L):
        h_p = _block(h_p, idx2e, dst2d, ea8, enc_mW1[l], enc_mb1[l],
                     enc_mW2[l], enc_mb2[l], enc_uW1[l], enc_ub1[l],
                     enc_uW2[l], enc_ub2[l])
    s1 = _fc1(h_p.reshape(1, N * F), fcE_W1, fcE_b1.reshape(1, FCH))
    t = _fc_mid(s1, fcE_W2, fcE_b2.reshape(1, LAT), fcD_W1,
                fcD_b1.reshape(1, FCH))
    d = _fc2(t, fcD_W2, fcD_b2.reshape(1, N * F))
    h_p = d.reshape(_NP, 128)
    for l in range(L):
        h_p = _block(h_p, idx2e, dst2d, ea8, dec_mW1[l], dec_mb1[l],
                     dec_mW2[l], dec_mb2[l], dec_uW1[l], dec_ub1[l],
                     dec_uW2[l], dec_ub2[l])
    return h_p.reshape(N, F)
